# baseline (speedup 1.0000x reference)
import functools
import jax
import jax.numpy as jnp
from jax import lax
from jax.experimental import pallas as pl
from jax.experimental.pallas import tpu as pltpu
from jax.experimental.pallas import tpu_sc as plsc

_N = 10000
_NP = 10240
_E = 320000
_EP = 327680
_EPT = _EP // 16
_DE = 16
_EPS = 1e-5
_HALF = _NP // 2
_CH = 160
_CHW = _CH * _NP
_NCH = _HALF // _CH
_TECW = _CHW // 16

_mesh = plsc.VectorSubcoreMesh(core_axis_name="c", subcore_axis_name="s")


@functools.partial(
    pl.kernel, mesh=_mesh,
    out_type=jax.ShapeDtypeStruct((_NP, _DE), jnp.float32),
    scratch_types=[
        pltpu.VMEM((2048,), jnp.int32),
        pltpu.VMEM((4, 128, _DE), jnp.float32),
        pltpu.VMEM((4, 128), jnp.int32),
        pltpu.VMEM((64,), jnp.int32),
        pltpu.VMEM_SHARED((5248, _DE), jnp.float32),
        pltpu.SemaphoreType.DMA,
        pltpu.SemaphoreType.DMA,
    ],
)
def _ea_kernel(dst_hbm, ea_hbm, zero_hbm, loop_hbm, out_hbm,
               dst_v, ea_ring, idx_ring, idx64_v, acc, sem_st, sem_sc):
    c = lax.axis_index("c")
    s = lax.axis_index("s")
    lo = c * _HALF
    ebase = s * _EPT

    pltpu.sync_copy(zero_hbm, acc.at[pl.ds(s * 328, 328)])
    plsc.subcore_barrier()

    def stage(st, carry):
        eb = ebase + st * 2048
        pltpu.sync_copy(dst_hbm.at[pl.ds(eb, 2048)], dst_v)

        def group(gr, carry2):
            st_copies = [
                pltpu.async_copy(
                    ea_hbm.at[pl.ds(eb + (gr * 4 + slot) * 128, 128)],
                    ea_ring.at[slot], sem_st)
                for slot in range(4)
            ]
            sc_copies = []
            for slot in range(4):
                st_copies[slot].wait()

                def vb(v, carry3, _slot=slot):
                    o = (gr * 4 + _slot) * 128 + v * 16
                    d = dst_v[pl.ds(o, 16)]
                    ok = (d >= lo) & (d < lo + _HALF)
                    idx_ring[_slot, pl.ds(v * 16, 16)] = jnp.where(
                        ok, d - lo, _HALF)
                    return carry3
                lax.fori_loop(0, 8, vb, 0)
                sc_copies.append(pltpu.async_copy(
                    ea_ring.at[slot], acc.at[idx_ring.at[slot]], sem_sc,
                    add=True))
            for cp in sc_copies:
                cp.wait()
            return carry2
        lax.fori_loop(0, 4, group, 0)
        return carry
    lax.fori_loop(0, 10, stage, 0)

    rbase = s * 320
    grow0 = c * _HALF + rbase
    lanes = lax.iota(jnp.int32, 16)
    pltpu.sync_copy(loop_hbm, ea_ring.at[0])

    def loop_idx(v, carry):
        g = grow0 + v * 16 + lanes
        ok = g < _N
        idx_ring[0, pl.ds(v * 16, 16)] = jnp.where(
            ok, rbase + v * 16 + lanes, _HALF)
        return carry
    lax.fori_loop(0, 8, loop_idx, 0)
    pltpu.sync_copy(ea_ring.at[0], acc.at[idx_ring.at[0]], add=True)

    def loop_idx2(v, carry):
        g = grow0 + 128 + v * 16 + lanes
        ok = g < _N
        idx_ring[0, pl.ds(v * 16, 16)] = jnp.where(
            ok, rbase + 128 + v * 16 + lanes, _HALF)
        return carry
    lax.fori_loop(0, 8, loop_idx2, 0)
    pltpu.sync_copy(ea_ring.at[0], acc.at[idx_ring.at[0]], add=True)

    def loop_idx3(v, carry):
        g = grow0 + 256 + v * 16 + lanes
        ok = g < _N
        idx64_v[pl.ds(v * 16, 16)] = jnp.where(
            ok, rbase + 256 + v * 16 + lanes, _HALF)
        return carry
    lax.fori_loop(0, 4, loop_idx3, 0)
    pltpu.sync_copy(ea_ring.at[0, pl.ds(0, 64)], acc.at[idx64_v], add=True)

    plsc.subcore_barrier()
    pltpu.sync_copy(acc.at[pl.ds(rbase, 320)],
                    out_hbm.at[pl.ds(c * _HALF + rbase, 320)])


@functools.partial(
    pl.kernel, mesh=_mesh,
    out_type=jax.ShapeDtypeStruct((_NP * _NP,), jnp.float32),
    scratch_types=[
        pltpu.VMEM((2048,), jnp.int32),
        pltpu.VMEM((2048,), jnp.int32),
        pltpu.VMEM((8, 128), jnp.int32),
        pltpu.VMEM((12800,), jnp.float32),
        pltpu.VMEM((128,), jnp.float32),
        pltpu.VMEM_SHARED((_CHW + 128,), jnp.float32),
        pltpu.SemaphoreType.DMA,
    ],
)
def _a_kernel(dst_hbm, src_hbm, zero_hbm, ones_hbm, out_hbm,
              dst_v, src_v, idx_ring, zero_v, ones_v, acc, sem):
    c = lax.axis_index("c")
    s = lax.axis_index("s")
    ebase = s * _EPT

    pltpu.sync_copy(zero_hbm, zero_v)
    pltpu.sync_copy(ones_hbm, ones_v)

    def chunk_body(ch, carry):
        gbase = (c * _HALF + ch * _CH) * _NP
        toff = s * _TECW

        def zb(z, carry2):
            pltpu.sync_copy(zero_v, acc.at[pl.ds(toff + z * 12800, 12800)])
            return carry2
        lax.fori_loop(0, 8, zb, 0)
        plsc.subcore_barrier()

        def stage(st, carry2):
            eb = ebase + st * 2048
            pltpu.sync_copy(dst_hbm.at[pl.ds(eb, 2048)], dst_v)
            pltpu.sync_copy(src_hbm.at[pl.ds(eb, 2048)], src_v)

            def group(gr, carry3):
                copies = []
                for slot in range(8):
                    def vb(v, carry4, _slot=slot):
                        o = (gr * 8 + _slot) * 128 + v * 16
                        d = dst_v[pl.ds(o, 16)]
                        sr = src_v[pl.ds(o, 16)]
                        li = d * _NP + sr - gbase
                        ok = (li >= 0) & (li < _CHW)
                        idx_ring[_slot, pl.ds(v * 16, 16)] = jnp.where(
                            ok, li, _CHW)
                        return carry4
                    lax.fori_loop(0, 8, vb, 0)
                    copies.append(pltpu.async_copy(
                        ones_v, acc.at[idx_ring.at[slot]], sem, add=True))
                for cp in copies:
                    cp.wait()
                return carry3
            lax.fori_loop(0, 2, group, 0)
            return carry2
        lax.fori_loop(0, 10, stage, 0)
        plsc.subcore_barrier()

        woff = gbase + s * _TECW
        pltpu.sync_copy(acc.at[pl.ds(toff, _TECW)],
                        out_hbm.at[pl.ds(woff, _TECW)])
        return carry
    lax.fori_loop(0, _NCH, chunk_body, 0)


def _prep_body(x_ref, ea_ref, wh_ref, we_ref, b_ref, u_ref, r_ref):
    u = jnp.dot(x_ref[...], wh_ref[...], preferred_element_type=jnp.float32)
    r = u + jnp.dot(ea_ref[...], we_ref[...],
                    preferred_element_type=jnp.float32) + b_ref[...]
    u_ref[...] = u.astype(jnp.bfloat16)
    r_ref[...] = r


def _prep(x, ea_sum, wh, we, b, bm=1024):
    np_, din = x.shape
    dh = wh.shape[1]
    grid = (np_ // bm,)
    return pl.pallas_call(
        _prep_body,
        grid=grid,
        in_specs=[
            pl.BlockSpec((bm, din), lambda i: (i, 0)),
            pl.BlockSpec((bm, _DE), lambda i: (i, 0)),
            pl.BlockSpec((din, dh), lambda i: (0, 0)),
            pl.BlockSpec((_DE, dh), lambda i: (0, 0)),
            pl.BlockSpec((1, dh), lambda i: (0, 0)),
        ],
        out_specs=[
            pl.BlockSpec((bm, dh), lambda i: (i, 0)),
            pl.BlockSpec((bm, dh), lambda i: (i, 0)),
        ],
        out_shape=[
            jax.ShapeDtypeStruct((np_, dh), jnp.bfloat16),
            jax.ShapeDtypeStruct((np_, dh), jnp.float32),
        ],
    )(x, ea_sum, wh, we, b.reshape(1, dh))


def _agg_body(a_ref, u_ref, r_ref, g_ref, be_ref, o_ref, acc_ref, *, nk):
    k = pl.program_id(1)

    @pl.when(k == 0)
    def _():
        acc_ref[...] = jnp.zeros_like(acc_ref)

    acc_ref[...] += jnp.dot(a_ref[...].astype(jnp.bfloat16), u_ref[...],
                            preferred_element_type=jnp.float32)

    @pl.when(k == nk - 1)
    def _():
        z = acc_ref[...] + r_ref[...]
        y = jnp.maximum(z, 0.0)
        mu = jnp.mean(y, axis=-1, keepdims=True)
        var = jnp.mean(jnp.square(y - mu), axis=-1, keepdims=True)
        yn = (y - mu) * jax.lax.rsqrt(var + _EPS) * g_ref[...] + be_ref[...]
        o_ref[...] = jnp.maximum(yn, 0.0)


def _agg_layer(a, u, r, g, be, bm=1024, bk=512):
    np_, dh = r.shape
    nk = np_ // bk
    grid = (np_ // bm, nk)
    return pl.pallas_call(
        functools.partial(_agg_body, nk=nk),
        grid=grid,
        in_specs=[
            pl.BlockSpec((bm, bk), lambda i, k: (i, k)),
            pl.BlockSpec((bk, dh), lambda i, k: (k, 0)),
            pl.BlockSpec((bm, dh), lambda i, k: (i, 0)),
            pl.BlockSpec((1, dh), lambda i, k: (0, 0)),
            pl.BlockSpec((1, dh), lambda i, k: (0, 0)),
        ],
        out_specs=pl.BlockSpec((bm, dh), lambda i, k: (i, 0)),
        out_shape=jax.ShapeDtypeStruct((np_, dh), jnp.float32),
        scratch_shapes=[pltpu.VMEM((bm, dh), jnp.float32)],
        compiler_params=pltpu.CompilerParams(
            dimension_semantics=("parallel", "arbitrary")),
    )(a, u, r, g.reshape(1, dh), be.reshape(1, dh))


def kernel(h, edge_index, edge_attr, W0, b0, W1, b1, W2, b2,
           g0, be0, g1, be1, g2, be2):
    src = edge_index[0]
    dst = edge_index[1]

    npad = _EP - _E
    dst_p = jnp.concatenate([dst, jnp.full((npad,), _NP, jnp.int32)])
    src_p = jnp.concatenate([src, jnp.zeros((npad,), jnp.int32)])
    ea_p = jnp.pad(edge_attr, ((0, npad), (0, 0)))

    zero_ea = jnp.zeros((328, _DE), jnp.float32)
    loop_rows = jnp.zeros((128, _DE), jnp.float32).at[:, _DE - 1].set(1.0)
    zero_a = jnp.zeros((12800,), jnp.float32)
    ones_a = jnp.ones((128,), jnp.float32)

    ea_sum = _ea_kernel(dst_p, ea_p, zero_ea, loop_rows)
    a = _a_kernel(dst_p, src_p, zero_a, ones_a).reshape(_NP, _NP)

    x = jnp.pad(h, ((0, _NP - _N), (0, 0)))
    for (w, b, g, be) in ((W0, b0, g0, be0), (W1, b1, g1, be1),
                          (W2, b2, g2, be2)):
        u, r = _prep(x, ea_sum, w[_DE:], w[:_DE], b)
        x = _agg_layer(a, u, r, g, be)
    return x[:_N]


# speedup vs baseline: 1.0009x; 1.0009x over previous
import functools
import jax
import jax.numpy as jnp
from jax import lax
from jax.experimental import pallas as pl
from jax.experimental.pallas import tpu as pltpu
from jax.experimental.pallas import tpu_sc as plsc

_N = 10000
_NP = 10240
_E = 320000
_EP = 327680
_EPT = _EP // 16
_DE = 16
_EPS = 1e-5
_HALF = _NP // 2
_CH = 160
_CHW = _CH * _NP
_NCH = _HALF // _CH
_TECW = _CHW // 16

_mesh = plsc.VectorSubcoreMesh(core_axis_name="c", subcore_axis_name="s")


@functools.partial(
    pl.kernel, mesh=_mesh,
    out_type=jax.ShapeDtypeStruct((_NP, _DE), jnp.float32),
    scratch_types=[
        pltpu.VMEM((2048,), jnp.int32),
        pltpu.VMEM((4, 128, _DE), jnp.float32),
        pltpu.VMEM((4, 128), jnp.int32),
        pltpu.VMEM((64,), jnp.int32),
        pltpu.VMEM_SHARED((5248, _DE), jnp.float32),
        pltpu.SemaphoreType.DMA,
        pltpu.SemaphoreType.DMA,
    ],
)
def _ea_kernel(dst_hbm, ea_hbm, zero_hbm, loop_hbm, out_hbm,
               dst_v, ea_ring, idx_ring, idx64_v, acc, sem_st, sem_sc):
    c = lax.axis_index("c")
    s = lax.axis_index("s")
    lo = c * _HALF
    ebase = s * _EPT

    pltpu.sync_copy(zero_hbm, acc.at[pl.ds(s * 328, 328)])
    plsc.subcore_barrier()

    def stage(st, carry):
        eb = ebase + st * 2048
        pltpu.sync_copy(dst_hbm.at[pl.ds(eb, 2048)], dst_v)

        def group(gr, carry2):
            st_copies = [
                pltpu.async_copy(
                    ea_hbm.at[pl.ds(eb + (gr * 4 + slot) * 128, 128)],
                    ea_ring.at[slot], sem_st)
                for slot in range(4)
            ]
            sc_copies = []
            for slot in range(4):
                st_copies[slot].wait()

                def vb(v, carry3, _slot=slot):
                    o = (gr * 4 + _slot) * 128 + v * 16
                    d = dst_v[pl.ds(o, 16)]
                    ok = (d >= lo) & (d < lo + _HALF)
                    idx_ring[_slot, pl.ds(v * 16, 16)] = jnp.where(
                        ok, d - lo, _HALF)
                    return carry3
                lax.fori_loop(0, 8, vb, 0)
                sc_copies.append(pltpu.async_copy(
                    ea_ring.at[slot], acc.at[idx_ring.at[slot]], sem_sc,
                    add=True))
            for cp in sc_copies:
                cp.wait()
            return carry2
        lax.fori_loop(0, 4, group, 0)
        return carry
    lax.fori_loop(0, 10, stage, 0)

    rbase = s * 320
    grow0 = c * _HALF + rbase
    lanes = lax.iota(jnp.int32, 16)
    pltpu.sync_copy(loop_hbm, ea_ring.at[0])

    def loop_idx(v, carry):
        g = grow0 + v * 16 + lanes
        ok = g < _N
        idx_ring[0, pl.ds(v * 16, 16)] = jnp.where(
            ok, rbase + v * 16 + lanes, _HALF)
        return carry
    lax.fori_loop(0, 8, loop_idx, 0)
    pltpu.sync_copy(ea_ring.at[0], acc.at[idx_ring.at[0]], add=True)

    def loop_idx2(v, carry):
        g = grow0 + 128 + v * 16 + lanes
        ok = g < _N
        idx_ring[0, pl.ds(v * 16, 16)] = jnp.where(
            ok, rbase + 128 + v * 16 + lanes, _HALF)
        return carry
    lax.fori_loop(0, 8, loop_idx2, 0)
    pltpu.sync_copy(ea_ring.at[0], acc.at[idx_ring.at[0]], add=True)

    def loop_idx3(v, carry):
        g = grow0 + 256 + v * 16 + lanes
        ok = g < _N
        idx64_v[pl.ds(v * 16, 16)] = jnp.where(
            ok, rbase + 256 + v * 16 + lanes, _HALF)
        return carry
    lax.fori_loop(0, 4, loop_idx3, 0)
    pltpu.sync_copy(ea_ring.at[0, pl.ds(0, 64)], acc.at[idx64_v], add=True)

    plsc.subcore_barrier()
    pltpu.sync_copy(acc.at[pl.ds(rbase, 320)],
                    out_hbm.at[pl.ds(c * _HALF + rbase, 320)])


@functools.partial(
    pl.kernel, mesh=_mesh,
    out_type=jax.ShapeDtypeStruct((_NP * _NP,), jnp.float32),
    scratch_types=[
        pltpu.VMEM((2048,), jnp.int32),
        pltpu.VMEM((2048,), jnp.int32),
        pltpu.VMEM((16, 128), jnp.int32),
        pltpu.VMEM((12800,), jnp.float32),
        pltpu.VMEM((128,), jnp.float32),
        pltpu.VMEM_SHARED((_CHW + 128,), jnp.float32),
        pltpu.SemaphoreType.DMA,
    ],
)
def _a_kernel(dst_hbm, src_hbm, zero_hbm, ones_hbm, out_hbm,
              dst_v, src_v, idx_ring, zero_v, ones_v, acc, sem):
    c = lax.axis_index("c")
    s = lax.axis_index("s")
    ebase = s * _EPT

    pltpu.sync_copy(zero_hbm, zero_v)
    pltpu.sync_copy(ones_hbm, ones_v)

    def chunk_body(ch, carry):
        gbase = (c * _HALF + ch * _CH) * _NP
        toff = s * _TECW

        zcopies = [
            pltpu.async_copy(zero_v,
                             acc.at[pl.ds(toff + z * 12800, 12800)], sem)
            for z in range(8)
        ]
        for cp in zcopies:
            cp.wait()
        plsc.subcore_barrier()

        def stage(st, carry2):
            eb = ebase + st * 2048
            pltpu.sync_copy(dst_hbm.at[pl.ds(eb, 2048)], dst_v)
            pltpu.sync_copy(src_hbm.at[pl.ds(eb, 2048)], src_v)

            copies = []
            for slot in range(16):
                def vb(v, carry4, _slot=slot):
                    o = _slot * 128 + v * 16
                    d = dst_v[pl.ds(o, 16)]
                    sr = src_v[pl.ds(o, 16)]
                    li = d * _NP + sr - gbase
                    ok = (li >= 0) & (li < _CHW)
                    idx_ring[_slot, pl.ds(v * 16, 16)] = jnp.where(
                        ok, li, _CHW)
                    return carry4
                lax.fori_loop(0, 8, vb, 0)
                copies.append(pltpu.async_copy(
                    ones_v, acc.at[idx_ring.at[slot]], sem, add=True))
            for cp in copies:
                cp.wait()
            return carry2
        lax.fori_loop(0, 10, stage, 0)
        plsc.subcore_barrier()

        woff = gbase + s * _TECW
        pltpu.sync_copy(acc.at[pl.ds(toff, _TECW)],
                        out_hbm.at[pl.ds(woff, _TECW)])
        return carry
    lax.fori_loop(0, _NCH, chunk_body, 0)


def _prep_body(x_ref, ea_ref, wh_ref, we_ref, b_ref, u_ref, r_ref):
    u = jnp.dot(x_ref[...], wh_ref[...], preferred_element_type=jnp.float32)
    r = u + jnp.dot(ea_ref[...], we_ref[...],
                    preferred_element_type=jnp.float32) + b_ref[...]
    u_ref[...] = u.astype(jnp.bfloat16)
    r_ref[...] = r


def _prep(x, ea_sum, wh, we, b, bm=1024):
    np_, din = x.shape
    dh = wh.shape[1]
    grid = (np_ // bm,)
    return pl.pallas_call(
        _prep_body,
        grid=grid,
        in_specs=[
            pl.BlockSpec((bm, din), lambda i: (i, 0)),
            pl.BlockSpec((bm, _DE), lambda i: (i, 0)),
            pl.BlockSpec((din, dh), lambda i: (0, 0)),
            pl.BlockSpec((_DE, dh), lambda i: (0, 0)),
            pl.BlockSpec((1, dh), lambda i: (0, 0)),
        ],
        out_specs=[
            pl.BlockSpec((bm, dh), lambda i: (i, 0)),
            pl.BlockSpec((bm, dh), lambda i: (i, 0)),
        ],
        out_shape=[
            jax.ShapeDtypeStruct((np_, dh), jnp.bfloat16),
            jax.ShapeDtypeStruct((np_, dh), jnp.float32),
        ],
    )(x, ea_sum, wh, we, b.reshape(1, dh))


def _agg_body(a_ref, u_ref, r_ref, g_ref, be_ref, o_ref, acc_ref, *, nk):
    k = pl.program_id(1)

    @pl.when(k == 0)
    def _():
        acc_ref[...] = jnp.zeros_like(acc_ref)

    acc_ref[...] += jnp.dot(a_ref[...].astype(jnp.bfloat16), u_ref[...],
                            preferred_element_type=jnp.float32)

    @pl.when(k == nk - 1)
    def _():
        z = acc_ref[...] + r_ref[...]
        y = jnp.maximum(z, 0.0)
        mu = jnp.mean(y, axis=-1, keepdims=True)
        var = jnp.mean(jnp.square(y - mu), axis=-1, keepdims=True)
        yn = (y - mu) * jax.lax.rsqrt(var + _EPS) * g_ref[...] + be_ref[...]
        o_ref[...] = jnp.maximum(yn, 0.0)


def _agg_layer(a, u, r, g, be, bm=1024, bk=512):
    np_, dh = r.shape
    nk = np_ // bk
    grid = (np_ // bm, nk)
    return pl.pallas_call(
        functools.partial(_agg_body, nk=nk),
        grid=grid,
        in_specs=[
            pl.BlockSpec((bm, bk), lambda i, k: (i, k)),
            pl.BlockSpec((bk, dh), lambda i, k: (k, 0)),
            pl.BlockSpec((bm, dh), lambda i, k: (i, 0)),
            pl.BlockSpec((1, dh), lambda i, k: (0, 0)),
            pl.BlockSpec((1, dh), lambda i, k: (0, 0)),
        ],
        out_specs=pl.BlockSpec((bm, dh), lambda i, k: (i, 0)),
        out_shape=jax.ShapeDtypeStruct((np_, dh), jnp.float32),
        scratch_shapes=[pltpu.VMEM((bm, dh), jnp.float32)],
        compiler_params=pltpu.CompilerParams(
            dimension_semantics=("parallel", "arbitrary")),
    )(a, u, r, g.reshape(1, dh), be.reshape(1, dh))


def kernel(h, edge_index, edge_attr, W0, b0, W1, b1, W2, b2,
           g0, be0, g1, be1, g2, be2):
    src = edge_index[0]
    dst = edge_index[1]

    npad = _EP - _E
    dst_p = jnp.concatenate([dst, jnp.full((npad,), _NP, jnp.int32)])
    src_p = jnp.concatenate([src, jnp.zeros((npad,), jnp.int32)])
    ea_p = jnp.pad(edge_attr, ((0, npad), (0, 0)))

    zero_ea = jnp.zeros((328, _DE), jnp.float32)
    loop_rows = jnp.zeros((128, _DE), jnp.float32).at[:, _DE - 1].set(1.0)
    zero_a = jnp.zeros((12800,), jnp.float32)
    ones_a = jnp.ones((128,), jnp.float32)

    ea_sum = _ea_kernel(dst_p, ea_p, zero_ea, loop_rows)
    a = _a_kernel(dst_p, src_p, zero_a, ones_a).reshape(_NP, _NP)

    x = jnp.pad(h, ((0, _NP - _N), (0, 0)))
    for (w, b, g, be) in ((W0, b0, g0, be0), (W1, b1, g1, be1),
                          (W2, b2, g2, be2)):
        u, r = _prep(x, ea_sum, w[_DE:], w[:_DE], b)
        x = _agg_layer(a, u, r, g, be)
    return x[:_N]


# speedup vs baseline: 3.5821x; 3.5790x over previous
import functools
import jax
import jax.numpy as jnp
from jax import lax
from jax.experimental import pallas as pl
from jax.experimental.pallas import tpu as pltpu
from jax.experimental.pallas import tpu_sc as plsc

_N = 10000
_NP = 10240
_E = 320000
_EP = 327680
_EPT = _EP // 16
_DE = 16
_EPS = 1e-5
_HALF = _NP // 2
_CH = 160
_CHW = _CH * _NP
_NCH = _HALF // _CH
_TECW = _CHW // 16

_mesh = plsc.VectorSubcoreMesh(core_axis_name="c", subcore_axis_name="s")


@functools.partial(
    pl.kernel, mesh=_mesh,
    out_type=jax.ShapeDtypeStruct((_NP, _DE), jnp.float32),
    scratch_types=[
        pltpu.VMEM((2048,), jnp.int32),
        pltpu.VMEM((4, 128, _DE), jnp.float32),
        pltpu.VMEM((4, 128), jnp.int32),
        pltpu.VMEM((64,), jnp.int32),
        pltpu.VMEM_SHARED((5248, _DE), jnp.float32),
        pltpu.SemaphoreType.DMA,
        pltpu.SemaphoreType.DMA,
    ],
)
def _ea_kernel(dst_hbm, ea_hbm, zero_hbm, loop_hbm, out_hbm,
               dst_v, ea_ring, idx_ring, idx64_v, acc, sem_st, sem_sc):
    c = lax.axis_index("c")
    s = lax.axis_index("s")
    lo = c * _HALF
    ebase = s * _EPT

    pltpu.sync_copy(zero_hbm, acc.at[pl.ds(s * 328, 328)])
    plsc.subcore_barrier()

    def stage(st, carry):
        eb = ebase + st * 2048
        pltpu.sync_copy(dst_hbm.at[pl.ds(eb, 2048)], dst_v)

        def group(gr, carry2):
            st_copies = [
                pltpu.async_copy(
                    ea_hbm.at[pl.ds(eb + (gr * 4 + slot) * 128, 128)],
                    ea_ring.at[slot], sem_st)
                for slot in range(4)
            ]
            sc_copies = []
            for slot in range(4):
                st_copies[slot].wait()

                def vb(v, carry3, _slot=slot):
                    o = (gr * 4 + _slot) * 128 + v * 16
                    d = dst_v[pl.ds(o, 16)]
                    ok = (d >= lo) & (d < lo + _HALF)
                    idx_ring[_slot, pl.ds(v * 16, 16)] = jnp.where(
                        ok, d - lo, _HALF)
                    return carry3
                lax.fori_loop(0, 8, vb, 0)
                sc_copies.append(pltpu.async_copy(
                    ea_ring.at[slot], acc.at[idx_ring.at[slot]], sem_sc,
                    add=True))
            for cp in sc_copies:
                cp.wait()
            return carry2
        lax.fori_loop(0, 4, group, 0)
        return carry
    lax.fori_loop(0, 10, stage, 0)

    rbase = s * 320
    grow0 = c * _HALF + rbase
    lanes = lax.iota(jnp.int32, 16)
    pltpu.sync_copy(loop_hbm, ea_ring.at[0])

    def loop_idx(v, carry):
        g = grow0 + v * 16 + lanes
        ok = g < _N
        idx_ring[0, pl.ds(v * 16, 16)] = jnp.where(
            ok, rbase + v * 16 + lanes, _HALF)
        return carry
    lax.fori_loop(0, 8, loop_idx, 0)
    pltpu.sync_copy(ea_ring.at[0], acc.at[idx_ring.at[0]], add=True)

    def loop_idx2(v, carry):
        g = grow0 + 128 + v * 16 + lanes
        ok = g < _N
        idx_ring[0, pl.ds(v * 16, 16)] = jnp.where(
            ok, rbase + 128 + v * 16 + lanes, _HALF)
        return carry
    lax.fori_loop(0, 8, loop_idx2, 0)
    pltpu.sync_copy(ea_ring.at[0], acc.at[idx_ring.at[0]], add=True)

    def loop_idx3(v, carry):
        g = grow0 + 256 + v * 16 + lanes
        ok = g < _N
        idx64_v[pl.ds(v * 16, 16)] = jnp.where(
            ok, rbase + 256 + v * 16 + lanes, _HALF)
        return carry
    lax.fori_loop(0, 4, loop_idx3, 0)
    pltpu.sync_copy(ea_ring.at[0, pl.ds(0, 64)], acc.at[idx64_v], add=True)

    plsc.subcore_barrier()
    pltpu.sync_copy(acc.at[pl.ds(rbase, 320)],
                    out_hbm.at[pl.ds(c * _HALF + rbase, 320)])


def _prep_body(x_ref, ea_ref, wh_ref, we_ref, b_ref, u_ref, r_ref):
    u = jnp.dot(x_ref[...], wh_ref[...], preferred_element_type=jnp.float32)
    r = u + jnp.dot(ea_ref[...], we_ref[...],
                    preferred_element_type=jnp.float32) + b_ref[...]
    u_ref[...] = u.astype(jnp.bfloat16)
    r_ref[...] = r


def _prep(x, ea_sum, wh, we, b, bm=1024):
    np_, din = x.shape
    dh = wh.shape[1]
    grid = (np_ // bm,)
    return pl.pallas_call(
        _prep_body,
        grid=grid,
        in_specs=[
            pl.BlockSpec((bm, din), lambda i: (i, 0)),
            pl.BlockSpec((bm, _DE), lambda i: (i, 0)),
            pl.BlockSpec((din, dh), lambda i: (0, 0)),
            pl.BlockSpec((_DE, dh), lambda i: (0, 0)),
            pl.BlockSpec((1, dh), lambda i: (0, 0)),
        ],
        out_specs=[
            pl.BlockSpec((bm, dh), lambda i: (i, 0)),
            pl.BlockSpec((bm, dh), lambda i: (i, 0)),
        ],
        out_shape=[
            jax.ShapeDtypeStruct((np_, dh), jnp.bfloat16),
            jax.ShapeDtypeStruct((np_, dh), jnp.float32),
        ],
    )(x, ea_sum, wh, we, b.reshape(1, dh))


def _agg_body(a_ref, u_ref, r_ref, g_ref, be_ref, o_ref, acc_ref, *, nk):
    k = pl.program_id(1)

    @pl.when(k == 0)
    def _():
        acc_ref[...] = jnp.zeros_like(acc_ref)

    acc_ref[...] += jnp.dot(a_ref[...].astype(jnp.bfloat16), u_ref[...],
                            preferred_element_type=jnp.float32)

    @pl.when(k == nk - 1)
    def _():
        z = acc_ref[...] + r_ref[...]
        y = jnp.maximum(z, 0.0)
        mu = jnp.mean(y, axis=-1, keepdims=True)
        var = jnp.mean(jnp.square(y - mu), axis=-1, keepdims=True)
        yn = (y - mu) * jax.lax.rsqrt(var + _EPS) * g_ref[...] + be_ref[...]
        o_ref[...] = jnp.maximum(yn, 0.0)


def _agg_layer(a, u, r, g, be, bm=1024, bk=512):
    np_, dh = r.shape
    nk = np_ // bk
    grid = (np_ // bm, nk)
    return pl.pallas_call(
        functools.partial(_agg_body, nk=nk),
        grid=grid,
        in_specs=[
            pl.BlockSpec((bm, bk), lambda i, k: (i, k)),
            pl.BlockSpec((bk, dh), lambda i, k: (k, 0)),
            pl.BlockSpec((bm, dh), lambda i, k: (i, 0)),
            pl.BlockSpec((1, dh), lambda i, k: (0, 0)),
            pl.BlockSpec((1, dh), lambda i, k: (0, 0)),
        ],
        out_specs=pl.BlockSpec((bm, dh), lambda i, k: (i, 0)),
        out_shape=jax.ShapeDtypeStruct((np_, dh), jnp.float32),
        scratch_shapes=[pltpu.VMEM((bm, dh), jnp.float32)],
        compiler_params=pltpu.CompilerParams(
            dimension_semantics=("parallel", "arbitrary")),
    )(a, u, r, g.reshape(1, dh), be.reshape(1, dh))


def kernel(h, edge_index, edge_attr, W0, b0, W1, b1, W2, b2,
           g0, be0, g1, be1, g2, be2):
    src = edge_index[0]
    dst = edge_index[1]

    npad = _EP - _E
    dst_p = jnp.concatenate([dst, jnp.full((npad,), _NP, jnp.int32)])
    src_p = jnp.concatenate([src, jnp.zeros((npad,), jnp.int32)])
    ea_p = jnp.pad(edge_attr, ((0, npad), (0, 0)))

    zero_ea = jnp.zeros((328, _DE), jnp.float32)
    loop_rows = jnp.zeros((128, _DE), jnp.float32).at[:, _DE - 1].set(1.0)

    ea_sum = _ea_kernel(dst_p, ea_p, zero_ea, loop_rows)
    a = jnp.zeros((_NP, _NP), jnp.bfloat16).at[dst, src].add(1.0)

    x = jnp.pad(h, ((0, _NP - _N), (0, 0)))
    for (w, b, g, be) in ((W0, b0, g0, be0), (W1, b1, g1, be1),
                          (W2, b2, g2, be2)):
        u, r = _prep(x, ea_sum, w[_DE:], w[:_DE], b)
        x = _agg_layer(a, u, r, g, be)
    return x[:_N]


# speedup vs baseline: 3.5827x; 1.0002x over previous
import functools
import jax
import jax.numpy as jnp
from jax import lax
from jax.experimental import pallas as pl
from jax.experimental.pallas import tpu as pltpu
from jax.experimental.pallas import tpu_sc as plsc

_N = 10000
_NP = 10240
_E = 320000
_EP = 327680
_EPT = _EP // 16
_DE = 16
_EPS = 1e-5
_HALF = _NP // 2
_CH = 160
_CHW = _CH * _NP
_NCH = _HALF // _CH
_TECW = _CHW // 16

_mesh = plsc.VectorSubcoreMesh(core_axis_name="c", subcore_axis_name="s")


@functools.partial(
    pl.kernel, mesh=_mesh,
    out_type=jax.ShapeDtypeStruct((_NP, _DE), jnp.float32),
    scratch_types=[
        pltpu.VMEM((2048,), jnp.int32),
        pltpu.VMEM((4, 128, _DE), jnp.float32),
        pltpu.VMEM((4, 128), jnp.int32),
        pltpu.VMEM((64,), jnp.int32),
        pltpu.VMEM_SHARED((5248, _DE), jnp.float32),
        pltpu.SemaphoreType.DMA,
        pltpu.SemaphoreType.DMA,
    ],
)
def _ea_kernel(dst_hbm, ea_hbm, zero_hbm, loop_hbm, out_hbm,
               dst_v, ea_ring, idx_ring, idx64_v, acc, sem_st, sem_sc):
    c = lax.axis_index("c")
    s = lax.axis_index("s")
    lo = c * _HALF
    ebase = s * _EPT

    pltpu.sync_copy(zero_hbm, acc.at[pl.ds(s * 328, 328)])
    plsc.subcore_barrier()

    def stage(st, carry):
        eb = ebase + st * 2048
        pltpu.sync_copy(dst_hbm.at[pl.ds(eb, 2048)], dst_v)

        def group(gr, carry2):
            st_copies = [
                pltpu.async_copy(
                    ea_hbm.at[pl.ds(eb + (gr * 4 + slot) * 128, 128)],
                    ea_ring.at[slot], sem_st)
                for slot in range(4)
            ]
            sc_copies = []
            for slot in range(4):
                st_copies[slot].wait()

                def vb(v, carry3, _slot=slot):
                    o = (gr * 4 + _slot) * 128 + v * 16
                    d = dst_v[pl.ds(o, 16)]
                    ok = (d >= lo) & (d < lo + _HALF)
                    idx_ring[_slot, pl.ds(v * 16, 16)] = jnp.where(
                        ok, d - lo, _HALF)
                    return carry3
                lax.fori_loop(0, 8, vb, 0)
                sc_copies.append(pltpu.async_copy(
                    ea_ring.at[slot], acc.at[idx_ring.at[slot]], sem_sc,
                    add=True))
            for cp in sc_copies:
                cp.wait()
            return carry2
        lax.fori_loop(0, 4, group, 0)
        return carry
    lax.fori_loop(0, 10, stage, 0)

    rbase = s * 320
    grow0 = c * _HALF + rbase
    lanes = lax.iota(jnp.int32, 16)
    pltpu.sync_copy(loop_hbm, ea_ring.at[0])

    def loop_idx(v, carry):
        g = grow0 + v * 16 + lanes
        ok = g < _N
        idx_ring[0, pl.ds(v * 16, 16)] = jnp.where(
            ok, rbase + v * 16 + lanes, _HALF)
        return carry
    lax.fori_loop(0, 8, loop_idx, 0)
    pltpu.sync_copy(ea_ring.at[0], acc.at[idx_ring.at[0]], add=True)

    def loop_idx2(v, carry):
        g = grow0 + 128 + v * 16 + lanes
        ok = g < _N
        idx_ring[0, pl.ds(v * 16, 16)] = jnp.where(
            ok, rbase + 128 + v * 16 + lanes, _HALF)
        return carry
    lax.fori_loop(0, 8, loop_idx2, 0)
    pltpu.sync_copy(ea_ring.at[0], acc.at[idx_ring.at[0]], add=True)

    def loop_idx3(v, carry):
        g = grow0 + 256 + v * 16 + lanes
        ok = g < _N
        idx64_v[pl.ds(v * 16, 16)] = jnp.where(
            ok, rbase + 256 + v * 16 + lanes, _HALF)
        return carry
    lax.fori_loop(0, 4, loop_idx3, 0)
    pltpu.sync_copy(ea_ring.at[0, pl.ds(0, 64)], acc.at[idx64_v], add=True)

    plsc.subcore_barrier()
    pltpu.sync_copy(acc.at[pl.ds(rbase, 320)],
                    out_hbm.at[pl.ds(c * _HALF + rbase, 320)])


def _prep_body(x_ref, ea_ref, wh_ref, we_ref, b_ref, u_ref, r_ref):
    u = jnp.dot(x_ref[...], wh_ref[...], preferred_element_type=jnp.float32)
    r = u + jnp.dot(ea_ref[...], we_ref[...],
                    preferred_element_type=jnp.float32) + b_ref[...]
    u_ref[...] = u.astype(jnp.bfloat16)
    r_ref[...] = r


def _prep(x, ea_sum, wh, we, b, bm=1024):
    np_, din = x.shape
    dh = wh.shape[1]
    grid = (np_ // bm,)
    return pl.pallas_call(
        _prep_body,
        grid=grid,
        in_specs=[
            pl.BlockSpec((bm, din), lambda i: (i, 0)),
            pl.BlockSpec((bm, _DE), lambda i: (i, 0)),
            pl.BlockSpec((din, dh), lambda i: (0, 0)),
            pl.BlockSpec((_DE, dh), lambda i: (0, 0)),
            pl.BlockSpec((1, dh), lambda i: (0, 0)),
        ],
        out_specs=[
            pl.BlockSpec((bm, dh), lambda i: (i, 0)),
            pl.BlockSpec((bm, dh), lambda i: (i, 0)),
        ],
        out_shape=[
            jax.ShapeDtypeStruct((np_, dh), jnp.bfloat16),
            jax.ShapeDtypeStruct((np_, dh), jnp.float32),
        ],
    )(x, ea_sum, wh, we, b.reshape(1, dh))


def _agg_body(a_ref, u_ref, r_ref, g_ref, be_ref, o_ref, acc_ref, *, nk):
    k = pl.program_id(1)

    @pl.when(k == 0)
    def _():
        acc_ref[...] = jnp.zeros_like(acc_ref)

    acc_ref[...] += jnp.dot(a_ref[...].astype(jnp.bfloat16), u_ref[...],
                            preferred_element_type=jnp.float32)

    @pl.when(k == nk - 1)
    def _():
        z = acc_ref[...] + r_ref[...]
        y = jnp.maximum(z, 0.0)
        mu = jnp.mean(y, axis=-1, keepdims=True)
        var = jnp.mean(jnp.square(y - mu), axis=-1, keepdims=True)
        yn = (y - mu) * jax.lax.rsqrt(var + _EPS) * g_ref[...] + be_ref[...]
        o_ref[...] = jnp.maximum(yn, 0.0)


def _agg_layer(a, u, r, g, be, bm=1024, bk=512):
    np_, dh = r.shape
    nk = np_ // bk
    grid = (np_ // bm, nk)
    return pl.pallas_call(
        functools.partial(_agg_body, nk=nk),
        grid=grid,
        in_specs=[
            pl.BlockSpec((bm, bk), lambda i, k: (i, k)),
            pl.BlockSpec((bk, dh), lambda i, k: (k, 0)),
            pl.BlockSpec((bm, dh), lambda i, k: (i, 0)),
            pl.BlockSpec((1, dh), lambda i, k: (0, 0)),
            pl.BlockSpec((1, dh), lambda i, k: (0, 0)),
        ],
        out_specs=pl.BlockSpec((bm, dh), lambda i, k: (i, 0)),
        out_shape=jax.ShapeDtypeStruct((np_, dh), jnp.float32),
        scratch_shapes=[pltpu.VMEM((bm, dh), jnp.float32)],
        compiler_params=pltpu.CompilerParams(
            dimension_semantics=("parallel", "arbitrary")),
    )(a, u, r, g.reshape(1, dh), be.reshape(1, dh))


def kernel(h, edge_index, edge_attr, W0, b0, W1, b1, W2, b2,
           g0, be0, g1, be1, g2, be2):
    src = edge_index[0]
    dst = edge_index[1]

    npad = _EP - _E
    dst_p = jnp.concatenate([dst, jnp.full((npad,), _NP, jnp.int32)])
    ea_p = jnp.pad(edge_attr, ((0, npad), (0, 0)))

    zero_ea = jnp.zeros((328, _DE), jnp.float32)
    loop_rows = jnp.zeros((128, _DE), jnp.float32).at[:, _DE - 1].set(1.0)

    ea_sum = _ea_kernel(dst_p, ea_p, zero_ea, loop_rows)
    a = jnp.zeros((_NP, _NP), jnp.bfloat16).at[dst, src].add(1.0)

    x = jnp.pad(h, ((0, _NP - _N), (0, 0)))
    for (w, b, g, be) in ((W0, b0, g0, be0), (W1, b1, g1, be1),
                          (W2, b2, g2, be2)):
        u, r = _prep(x, ea_sum, w[_DE:], w[:_DE], b)
        x = _agg_layer(a, u, r, g, be)
    return x[:_N]
